# baseline (speedup 1.0000x reference)
import functools
import math

import jax
import jax.numpy as jnp
from jax import lax
from jax.experimental import pallas as pl
from jax.experimental.pallas import tpu as pltpu
from jax.experimental.pallas import tpu_sc as plsc

D = 64
V = 1000000
SCALE = 8.0

NC = 2
NS = 16
NW = NC * NS

IDX_MINOR = 128
K = 2
CHUNK = K * IDX_MINOR
LANES = 16

NBUF = 5
PREFETCH = 3

TCB = 512
TB = 8
SB = 128


def _tc_pack(lutT):
    nblk = (V + TCB - 1) // TCB

    def body(x_ref, y_ref):
        x = x_ref[...]
        x3 = x.reshape(D, TCB // 2, 2)
        e = x3[:, :, 0]
        o = x3[:, :, 1]
        y_ref[...] = jnp.concatenate([e.T, o.T], axis=1)

    return pl.pallas_call(
        body,
        grid=(nblk,),
        in_specs=[pl.BlockSpec((D, TCB), lambda i: (0, i))],
        out_specs=pl.BlockSpec((TCB // 2, 2 * D), lambda i: (i, 0)),
        out_shape=jax.ShapeDtypeStruct((V // 2, 2 * D), jnp.float32),
    )(lutT)


def _tc_untranspose(z2, S, T):
    def body(x_ref, y_ref):
        x = x_ref[...]
        y = x.T
        y_ref[...] = y.reshape(TB, D, SB)

    return pl.pallas_call(
        body,
        grid=(T // TB, S // SB),
        in_specs=[pl.BlockSpec((SB, TB * D), lambda t, s: (s, t))],
        out_specs=pl.BlockSpec((TB, D, SB), lambda t, s: (t, 0, s)),
        out_shape=jax.ShapeDtypeStruct((T, D, S), jnp.float32),
    )(z2)


def _sc_gather(idx2d, tab):
    n_idx_rows, _ = idx2d.shape
    B = n_idx_rows * IDX_MINOR
    b_per_w = B // NW
    n_chunks = b_per_w // CHUNK
    idx_rows_per_w = b_per_w // IDX_MINOR
    assert n_chunks % NBUF == 0

    mesh = plsc.VectorSubcoreMesh(core_axis_name="c", subcore_axis_name="s")

    @functools.partial(
        pl.kernel,
        out_type=jax.ShapeDtypeStruct((B, D), jnp.float32),
        mesh=mesh,
        scratch_types=[
            pltpu.VMEM((idx_rows_per_w, IDX_MINOR), jnp.int32),
            pltpu.VMEM((NBUF, CHUNK, D), jnp.float32),
            pltpu.SemaphoreType.DMA((NBUF,)),
            pltpu.SemaphoreType.DMA((NBUF,)),
        ],
        compiler_params=pltpu.CompilerParams(use_tc_tiling_on_sc=False),
    )
    def k(idx_hbm, table_hbm, out_hbm, idx_v, rows_v, gsem, osem):
        wid = lax.axis_index("s") * NC + lax.axis_index("c")
        row_base = pl.multiple_of(wid * b_per_w, 8)
        idx_row_base = pl.multiple_of(wid * idx_rows_per_w, 8)
        pltpu.sync_copy(idx_hbm.at[pl.ds(idx_row_base, idx_rows_per_w)], idx_v)

        def fire_gather(f, b):
            for jj in range(K):
                pltpu.async_copy(
                    table_hbm.at[idx_v.at[f * K + jj]],
                    rows_v.at[b, pl.ds(jj * IDX_MINOR, IDX_MINOR)],
                    gsem.at[b],
                )

        def wait_gather(b):
            pltpu.make_async_copy(
                table_hbm.at[pl.ds(0, CHUNK)], rows_v.at[b], gsem.at[b]
            ).wait()

        def wait_out(b):
            pltpu.make_async_copy(
                rows_v.at[b], out_hbm.at[pl.ds(0, CHUNK)], osem.at[b]
            ).wait()

        for p in range(PREFETCH):
            fire_gather(p, p)

        def super_body(s, carry):
            g0 = s * NBUF
            for j in range(NBUF):
                g = g0 + j
                f = g + PREFETCH
                bf = (j + PREFETCH) % NBUF

                @pl.when(jnp.logical_and(f >= NBUF, f < n_chunks))
                def _():
                    wait_out(bf)

                @pl.when(f < n_chunks)
                def _():
                    fire_gather(f, bf)

                wait_gather(j)

                def scale_row(i, c):
                    for q in range(D // LANES):
                        sl = pl.ds(q * LANES, LANES)
                        rows_v[j, i, sl] = rows_v[j, i, sl] * SCALE
                    return c

                lax.fori_loop(0, CHUNK, scale_row, 0)

                row0 = pl.multiple_of(row_base + g * CHUNK, 8)
                pltpu.async_copy(
                    rows_v.at[j], out_hbm.at[pl.ds(row0, CHUNK)], osem.at[j]
                )
            return carry

        lax.fori_loop(0, n_chunks // NBUF, super_body, 0)

        for j in range(NBUF):
            wait_out(j)

    return k(idx2d, tab)


def kernel(x, lut):
    S, T = x.shape
    B = S * T
    lutT = lut.T
    packed = _tc_pack(lutT)
    tab = packed.reshape(V, D)
    idx2d = x.reshape(B // IDX_MINOR, IDX_MINOR)
    z = _sc_gather(idx2d, tab)
    z2 = z.reshape(S, T * D)
    y = _tc_untranspose(z2, S, T)
    return jnp.transpose(y, (2, 0, 1))


# speedup vs baseline: 11.5737x; 11.5737x over previous
import functools
import math

import jax
import jax.numpy as jnp
from jax import lax
from jax.experimental import pallas as pl
from jax.experimental.pallas import tpu as pltpu
from jax.experimental.pallas import tpu_sc as plsc

D = 64
SCALE = 8.0

NC = 2
NS = 16
NW = NC * NS

IDX_MINOR = 128
K = 2
CHUNK = K * IDX_MINOR
LANES = 16

NBUF = 5
PREFETCH = 3


def _sc_gather(idx2d, tab):
    n_idx_rows, _ = idx2d.shape
    B = n_idx_rows * IDX_MINOR
    b_per_w = B // NW
    n_chunks = b_per_w // CHUNK
    idx_rows_per_w = b_per_w // IDX_MINOR
    assert n_chunks % NBUF == 0

    mesh = plsc.VectorSubcoreMesh(core_axis_name="c", subcore_axis_name="s")

    @functools.partial(
        pl.kernel,
        out_type=jax.ShapeDtypeStruct((B, D), jnp.float32),
        mesh=mesh,
        scratch_types=[
            pltpu.VMEM((idx_rows_per_w, IDX_MINOR), jnp.int32),
            pltpu.VMEM((NBUF, CHUNK, D), jnp.float32),
            pltpu.SemaphoreType.DMA((NBUF,)),
            pltpu.SemaphoreType.DMA((NBUF,)),
        ],
        compiler_params=pltpu.CompilerParams(use_tc_tiling_on_sc=False),
    )
    def k(idx_hbm, table_hbm, out_hbm, idx_v, rows_v, gsem, osem):
        wid = lax.axis_index("s") * NC + lax.axis_index("c")
        row_base = pl.multiple_of(wid * b_per_w, 8)
        idx_row_base = pl.multiple_of(wid * idx_rows_per_w, 8)
        pltpu.sync_copy(idx_hbm.at[pl.ds(idx_row_base, idx_rows_per_w)], idx_v)

        def fire_gather(f, b):
            for jj in range(K):
                pltpu.async_copy(
                    table_hbm.at[idx_v.at[f * K + jj]],
                    rows_v.at[b, pl.ds(jj * IDX_MINOR, IDX_MINOR)],
                    gsem.at[b],
                )

        def wait_gather(b):
            pltpu.make_async_copy(
                table_hbm.at[pl.ds(0, CHUNK)], rows_v.at[b], gsem.at[b]
            ).wait()

        def wait_out(b):
            pltpu.make_async_copy(
                rows_v.at[b], out_hbm.at[pl.ds(0, CHUNK)], osem.at[b]
            ).wait()

        for p in range(PREFETCH):
            fire_gather(p, p)

        def super_body(s, carry):
            g0 = s * NBUF
            for j in range(NBUF):
                g = g0 + j
                f = g + PREFETCH
                bf = (j + PREFETCH) % NBUF

                @pl.when(jnp.logical_and(f >= NBUF, f < n_chunks))
                def _():
                    wait_out(bf)

                @pl.when(f < n_chunks)
                def _():
                    fire_gather(f, bf)

                wait_gather(j)

                def scale_row(i, c):
                    for q in range(D // LANES):
                        sl = pl.ds(q * LANES, LANES)
                        rows_v[j, i, sl] = rows_v[j, i, sl] * SCALE
                    return c

                lax.fori_loop(0, CHUNK, scale_row, 0)

                row0 = pl.multiple_of(row_base + g * CHUNK, 8)
                pltpu.async_copy(
                    rows_v.at[j], out_hbm.at[pl.ds(row0, CHUNK)], osem.at[j]
                )
            return carry

        lax.fori_loop(0, n_chunks // NBUF, super_body, 0)

        for j in range(NBUF):
            wait_out(j)

    return k(idx2d, tab)


def kernel(x, lut):
    S, T = x.shape
    B = S * T
    idx2d = x.reshape(B // IDX_MINOR, IDX_MINOR)
    z = _sc_gather(idx2d, lut)
    return z.reshape(S, T, D)
